# baseline (speedup 1.0000x reference)
import jax
import jax.numpy as jnp
from jax.experimental import pallas as pl

NODE_N = {"tad": 10000, "atac_region": 50000, "gene": 20000, "protein": 20000, "gene_name": 20000}
D = 256
RELS = [
    ("tad", "overlaps", "atac_region", 160000),
    ("atac_region", "rev_overlaps", "tad", 160000),
    ("tad", "overlaps", "gene", 80000),
    ("gene", "rev_overlaps", "tad", 80000),
    ("atac_region", "overlaps", "gene", 160000),
    ("gene", "rev_overlaps", "atac_region", 160000),
    ("protein", "coexpressed", "protein", 320000),
    ("protein", "tf_interacts", "gene", 80000),
    ("gene", "rev_tf_interacts", "protein", 80000),
    ("protein", "rev_associated", "gene", 80000),
    ("gene", "associated", "protein", 80000),
]
NAME_REL = ("protein", "is_named", "gene_name", 20000)
N_LAYERS = 4
NODE_TYPES = ["tad", "atac_region", "gene", "protein"]


def _rk(r):
    return r[0] + "__" + r[1] + "__" + r[2]


def _sage(x_src, x_dst, ei, p):
    src, dst = ei[0], ei[1]
    msg = jnp.take(x_src, src, axis=0)
    n_dst = x_dst.shape[0]
    summed = jax.ops.segment_sum(msg, dst, num_segments=n_dst)
    cnt = jax.ops.segment_sum(jnp.ones((ei.shape[1],), x_src.dtype), dst, num_segments=n_dst)
    mean = summed / jnp.maximum(cnt, 1.0)[:, None]
    return mean @ p["W_l"] + p["b_l"] + x_dst @ p["W_r"]


def _identity_kernel(x_ref, o_ref):
    o_ref[...] = x_ref[...]


def _pallas_identity(x):
    return pl.pallas_call(
        _identity_kernel,
        out_shape=jax.ShapeDtypeStruct(x.shape, x.dtype),
    )(x)


def kernel(x_tad, x_atac_region, x_gene, x_protein, edge_index_tad__overlaps__atac_region, edge_index_atac_region__rev_overlaps__tad, edge_index_tad__overlaps__gene, edge_index_gene__rev_overlaps__tad, edge_index_atac_region__overlaps__gene, edge_index_gene__rev_overlaps__atac_region, edge_index_protein__coexpressed__protein, edge_index_protein__tf_interacts__gene, edge_index_gene__rev_tf_interacts__protein, edge_index_protein__rev_associated__gene, edge_index_gene__associated__protein, edge_index_protein__is_named__gene_name, params):
    kw = dict(locals())
    xs = {nt: kw["x_" + nt] for nt in NODE_TYPES}
    eis = {_rk(r): kw["edge_index_" + _rk(r)] for r in RELS + [NAME_REL]}
    x = dict(xs)
    for l in range(N_LAYERS):
        lp = params["conv%d" % l]
        new = {}
        for r in RELS:
            src, _, dst, _ = r
            k = _rk(r)
            o = _sage(x[src], x[dst], eis[k], lp[k])
            new[dst] = o if dst not in new else new[dst] + o
        x = {k2: jax.nn.relu(v) for k2, v in new.items()}
    x_gn = jnp.full((NODE_N["gene_name"], 1), -1.0, dtype=jnp.float32)
    ei_n = eis[_rk(NAME_REL)]
    gene_pred = _sage(x["protein"], x_gn, ei_n, params["name_conv"])
    zero_prob = jax.nn.sigmoid(_sage(x["protein"], x_gn, ei_n, params["zero_conv"]))
    return _pallas_identity(jnp.concatenate([gene_pred, zero_prob], axis=1))


# speedup vs baseline: 1.2512x; 1.2512x over previous
import functools

import jax
import jax.numpy as jnp
from jax import lax
from jax.experimental import pallas as pl
from jax.experimental.pallas import tpu as pltpu
from jax.experimental.pallas import tpu_sc as plsc

NODE_N = {"tad": 10000, "atac_region": 50000, "gene": 20000, "protein": 20000, "gene_name": 20000}
D = 256
H = 128
RELS = [
    ("tad", "overlaps", "atac_region", 160000),
    ("atac_region", "rev_overlaps", "tad", 160000),
    ("tad", "overlaps", "gene", 80000),
    ("gene", "rev_overlaps", "tad", 80000),
    ("atac_region", "overlaps", "gene", 160000),
    ("gene", "rev_overlaps", "atac_region", 160000),
    ("protein", "coexpressed", "protein", 320000),
    ("protein", "tf_interacts", "gene", 80000),
    ("gene", "rev_tf_interacts", "protein", 80000),
    ("protein", "rev_associated", "gene", 80000),
    ("gene", "associated", "protein", 80000),
]
NAME_REL = ("protein", "is_named", "gene_name", 20000)
N_LAYERS = 4
NODE_TYPES = ["tad", "atac_region", "gene", "protein"]

NC = 2
NS = 16
WD = 8192
CHUNK = 256
ACC_ROWS = WD + 128
RPT = WD // NS
ZR = ACC_ROWS // NS


def _rk(r):
    return r[0] + "__" + r[1] + "__" + r[2]


def _ceil(a, b):
    return -(-a // b)


def _bin_edges(src, dst, n_dst, n_src):
    E = src.shape[0]
    NW = _ceil(n_dst, WD)
    w = dst // WD
    cnts = jnp.zeros((NW,), jnp.int32).at[w].add(1)
    nch = (cnts + (CHUNK - 1)) // CHUNK
    basech = jnp.cumsum(nch) - nch
    base_cnt = jnp.cumsum(cnts) - cnts
    order = jnp.argsort(w, stable=True)
    w_s = w[order]
    pos = basech[w_s] * CHUNK + (jnp.arange(E, dtype=jnp.int32) - base_cnt[w_s])
    E_pad = E + NW * CHUNK
    ar = jnp.arange(E_pad, dtype=jnp.int32)
    srcs_b = ((ar * 97) % n_src).at[pos].set(src[order])
    ldst_b = (WD + (ar % 128)).at[pos].set(dst[order] - w_s * WD)
    nch16 = jnp.zeros((16,), jnp.int32).at[:NW].set(nch)
    basech16 = jnp.zeros((16,), jnp.int32).at[:NW].set(basech)
    return srcs_b, ldst_b, nch16, basech16, NW


@functools.cache
def _segsum_call(n_src, E_pad, NW):
    out_rows = NW * WD
    mesh = plsc.VectorSubcoreMesh(core_axis_name="c", subcore_axis_name="s")

    @functools.partial(
        pl.kernel,
        out_type=jax.ShapeDtypeStruct((out_rows, H), jnp.float32),
        mesh=mesh,
        compiler_params=pltpu.CompilerParams(needs_layout_passes=False),
        scratch_types=[
            pltpu.VMEM((16,), jnp.int32),
            pltpu.VMEM((16,), jnp.int32),
            pltpu.VMEM((CHUNK,), jnp.int32),
            pltpu.VMEM((CHUNK,), jnp.int32),
            pltpu.VMEM((CHUNK, H), jnp.float32),
            pltpu.VMEM((64, H), jnp.float32),
            pltpu.VMEM_SHARED((ACC_ROWS, H), jnp.float32),
            pltpu.SemaphoreType.DMA,
        ],
    )
    def call(x_hbm, srcs_hbm, ldst_hbm, nch_hbm, basech_hbm, zeros_hbm, out_hbm,
             nch_v, basech_v, sidx_v, ldst_v, rows_v, zbuf_v, acc, sem):
        cid = lax.axis_index("c")
        sid = lax.axis_index("s")
        pltpu.sync_copy(zeros_hbm, zbuf_v)
        pltpu.sync_copy(nch_hbm, nch_v)
        pltpu.sync_copy(basech_hbm, basech_v)
        nch = nch_v[...]
        basech = basech_v[...]
        lanes = lax.broadcasted_iota(jnp.int32, (16,), 0)
        for w in range(NW):
            @pl.when(w % NC == cid)
            def _():
                z0 = sid * ZR
                for b in range(ZR // 64):
                    pltpu.sync_copy(zbuf_v, acc.at[pl.ds(z0 + b * 64, 64)])
                rem = ZR % 64
                if rem:
                    pltpu.sync_copy(zbuf_v.at[pl.ds(0, rem)],
                                    acc.at[pl.ds(z0 + (ZR // 64) * 64, rem)])
                plsc.subcore_barrier()
                nw = jnp.sum(jnp.where(lanes == w, nch, 0))
                b0 = jnp.sum(jnp.where(lanes == w, basech, 0))
                trips = (nw - sid + NS - 1) // NS

                def body(j, carry):
                    ch = sid + j * NS
                    off = (b0 + ch) * CHUNK
                    pltpu.sync_copy(srcs_hbm.at[pl.ds(off, CHUNK)], sidx_v)
                    pltpu.sync_copy(ldst_hbm.at[pl.ds(off, CHUNK)], ldst_v)
                    pltpu.async_copy(x_hbm.at[sidx_v], rows_v, sem).wait()
                    pltpu.sync_copy(rows_v, acc.at[ldst_v], add=True)
                    return carry

                lax.fori_loop(0, trips, body, 0)
                plsc.subcore_barrier()
                pltpu.sync_copy(acc.at[pl.ds(sid * RPT, RPT)],
                                out_hbm.at[pl.ds(w * WD + sid * RPT, RPT)])
                plsc.subcore_barrier()

    return call


@functools.cache
def _counts_call(E_pad, NW):
    out_rows = NW * WD
    mesh = plsc.VectorSubcoreMesh(core_axis_name="c", subcore_axis_name="s")

    @functools.partial(
        pl.kernel,
        out_type=jax.ShapeDtypeStruct((out_rows, H), jnp.float32),
        mesh=mesh,
        compiler_params=pltpu.CompilerParams(needs_layout_passes=False),
        scratch_types=[
            pltpu.VMEM((16,), jnp.int32),
            pltpu.VMEM((16,), jnp.int32),
            pltpu.VMEM((CHUNK,), jnp.int32),
            pltpu.VMEM((CHUNK, H), jnp.float32),
            pltpu.VMEM((64, H), jnp.float32),
            pltpu.VMEM_SHARED((ACC_ROWS, H), jnp.float32),
        ],
    )
    def call(ldst_hbm, nch_hbm, basech_hbm, ones_hbm, zeros_hbm, out_hbm,
             nch_v, basech_v, ldst_v, ones_v, zbuf_v, acc):
        cid = lax.axis_index("c")
        sid = lax.axis_index("s")
        pltpu.sync_copy(ones_hbm, ones_v)
        pltpu.sync_copy(zeros_hbm, zbuf_v)
        pltpu.sync_copy(nch_hbm, nch_v)
        pltpu.sync_copy(basech_hbm, basech_v)
        nch = nch_v[...]
        basech = basech_v[...]
        lanes = lax.broadcasted_iota(jnp.int32, (16,), 0)
        for w in range(NW):
            @pl.when(w % NC == cid)
            def _():
                z0 = sid * ZR
                for b in range(ZR // 64):
                    pltpu.sync_copy(zbuf_v, acc.at[pl.ds(z0 + b * 64, 64)])
                rem = ZR % 64
                if rem:
                    pltpu.sync_copy(zbuf_v.at[pl.ds(0, rem)],
                                    acc.at[pl.ds(z0 + (ZR // 64) * 64, rem)])
                plsc.subcore_barrier()
                nw = jnp.sum(jnp.where(lanes == w, nch, 0))
                b0 = jnp.sum(jnp.where(lanes == w, basech, 0))
                trips = (nw - sid + NS - 1) // NS

                def body(j, carry):
                    ch = sid + j * NS
                    off = (b0 + ch) * CHUNK
                    pltpu.sync_copy(ldst_hbm.at[pl.ds(off, CHUNK)], ldst_v)
                    pltpu.sync_copy(ones_v, acc.at[ldst_v], add=True)
                    return carry

                lax.fori_loop(0, trips, body, 0)
                plsc.subcore_barrier()
                pltpu.sync_copy(acc.at[pl.ds(sid * RPT, RPT)],
                                out_hbm.at[pl.ds(w * WD + sid * RPT, RPT)])
                plsc.subcore_barrier()

    return call


@functools.cache
def _conv_call(n, k, bm=400):
    grid = (n // bm,)

    def body(*refs):
        w_ref = refs[2 + 3 * k]
        b_ref = refs[3 + 3 * k]
        olo_ref = refs[4 + 3 * k]
        ohi_ref = refs[5 + 3 * k]
        x = jnp.concatenate([refs[0][...], refs[1][...]], axis=1)
        acc = jnp.dot(x, w_ref[0:D, :], preferred_element_type=jnp.float32)
        for i in range(k):
            seg = jnp.concatenate([refs[2 + 3 * i][...], refs[3 + 3 * i][...]],
                                  axis=1)
            cnt = refs[4 + 3 * i][...]
            scale = 1.0 / jnp.maximum(cnt[:, 0:1], 1.0)
            acc = acc + jnp.dot(seg * scale, w_ref[D * (i + 1):D * (i + 2), :],
                                preferred_element_type=jnp.float32)
        out = jnp.maximum(acc + b_ref[...], 0.0)
        olo_ref[...] = out[:, 0:H]
        ohi_ref[...] = out[:, H:D]

    in_specs = [pl.BlockSpec((bm, H), lambda i: (i, 0)),
                pl.BlockSpec((bm, H), lambda i: (i, 0))]
    for _ in range(k):
        in_specs.append(pl.BlockSpec((bm, H), lambda i: (i, 0)))
        in_specs.append(pl.BlockSpec((bm, H), lambda i: (i, 0)))
        in_specs.append(pl.BlockSpec((bm, H), lambda i: (i, 0)))
    in_specs.append(pl.BlockSpec((D * (k + 1), D), lambda i: (0, 0)))
    in_specs.append(pl.BlockSpec((1, D), lambda i: (0, 0)))

    return pl.pallas_call(
        body,
        grid=grid,
        in_specs=in_specs,
        out_specs=[pl.BlockSpec((bm, H), lambda i: (i, 0)),
                   pl.BlockSpec((bm, H), lambda i: (i, 0))],
        out_shape=[jax.ShapeDtypeStruct((n, H), jnp.float32),
                   jax.ShapeDtypeStruct((n, H), jnp.float32)],
    )


@functools.cache
def _head_call(n, bm=1000):
    grid = (n // bm,)

    def body(slo_ref, shi_ref, cnt_ref, w_ref, b_ref, o_ref):
        scale = 1.0 / jnp.maximum(cnt_ref[...][:, 0:1], 1.0)
        seg = jnp.concatenate([slo_ref[...], shi_ref[...]], axis=1)
        raw = jnp.dot(seg * scale, w_ref[...],
                      preferred_element_type=jnp.float32) + b_ref[...]
        lane = lax.broadcasted_iota(jnp.int32, (bm, 128), 1)
        o_ref[...] = jnp.where(lane == 1, jax.nn.sigmoid(raw), raw)

    return pl.pallas_call(
        body,
        grid=grid,
        in_specs=[
            pl.BlockSpec((bm, H), lambda i: (i, 0)),
            pl.BlockSpec((bm, H), lambda i: (i, 0)),
            pl.BlockSpec((bm, H), lambda i: (i, 0)),
            pl.BlockSpec((D, 128), lambda i: (0, 0)),
            pl.BlockSpec((1, 128), lambda i: (0, 0)),
        ],
        out_specs=pl.BlockSpec((bm, 128), lambda i: (i, 0)),
        out_shape=jax.ShapeDtypeStruct((n, 128), jnp.float32),
    )


def kernel(x_tad, x_atac_region, x_gene, x_protein, edge_index_tad__overlaps__atac_region, edge_index_atac_region__rev_overlaps__tad, edge_index_tad__overlaps__gene, edge_index_gene__rev_overlaps__tad, edge_index_atac_region__overlaps__gene, edge_index_gene__rev_overlaps__atac_region, edge_index_protein__coexpressed__protein, edge_index_protein__tf_interacts__gene, edge_index_gene__rev_tf_interacts__protein, edge_index_protein__rev_associated__gene, edge_index_gene__associated__protein, edge_index_protein__is_named__gene_name, params):
    kw = dict(locals())
    xs = {nt: kw["x_" + nt] for nt in NODE_TYPES}
    eis = {_rk(r): kw["edge_index_" + _rk(r)] for r in RELS + [NAME_REL]}

    zerosH = jnp.zeros((64, H), jnp.float32)
    onesH = jnp.ones((CHUNK, H), jnp.float32)

    bins, cnts = {}, {}
    for r in RELS + [NAME_REL]:
        k = _rk(r)
        ei = eis[k]
        srcs_b, ldst_b, nch16, basech16, NW = _bin_edges(
            ei[0], ei[1], NODE_N[r[2]], NODE_N[r[0]])
        bins[k] = (srcs_b, ldst_b, nch16, basech16, NW)
        cnts[k] = _counts_call(srcs_b.shape[0], NW)(
            ldst_b, nch16, basech16, onesH, zerosH)

    x = {t: (xs[t][:, 0:H], xs[t][:, H:D]) for t in NODE_TYPES}
    for l in range(N_LAYERS):
        lp = params["conv%d" % l]
        segs = {}
        for r in RELS:
            k = _rk(r)
            srcs_b, ldst_b, nch16, basech16, NW = bins[k]
            call = _segsum_call(NODE_N[r[0]], srcs_b.shape[0], NW)
            segs[k] = tuple(
                call(x[r[0]][h], srcs_b, ldst_b, nch16, basech16, zerosH)
                for h in range(2))
        new = {}
        for dst_t in NODE_TYPES:
            rels_t = [r for r in RELS if r[2] == dst_t]
            ks = [_rk(r) for r in rels_t]
            wr_sum = sum(lp[k]["W_r"] for k in ks)
            wcat = jnp.concatenate([wr_sum] + [lp[k]["W_l"] for k in ks], axis=0)
            bias = sum(lp[k]["b_l"] for k in ks).reshape(1, D)
            n = NODE_N[dst_t]
            args = [x[dst_t][0], x[dst_t][1]]
            for k in ks:
                args.append(segs[k][0])
                args.append(segs[k][1])
                args.append(cnts[k])
            new[dst_t] = tuple(_conv_call(n, len(ks))(*args, wcat, bias))
        x = new

    kn = _rk(NAME_REL)
    srcs_b, ldst_b, nch16, basech16, NW = bins[kn]
    call = _segsum_call(NODE_N["protein"], srcs_b.shape[0], NW)
    seg_n = tuple(
        call(x["protein"][h], srcs_b, ldst_b, nch16, basech16, zerosH)
        for h in range(2))
    p1, p2 = params["name_conv"], params["zero_conv"]
    w2 = jnp.zeros((D, 128), jnp.float32)
    w2 = w2.at[:, 0].set(p1["W_l"][:, 0]).at[:, 1].set(p2["W_l"][:, 0])
    b2 = jnp.zeros((1, 128), jnp.float32)
    b2 = b2.at[0, 0].set(p1["b_l"][0] - p1["W_r"][0, 0])
    b2 = b2.at[0, 1].set(p2["b_l"][0] - p2["W_r"][0, 0])
    outh = _head_call(NODE_N["gene_name"])(seg_n[0], seg_n[1], cnts[kn], w2, b2)
    return outh[:, :2]


# speedup vs baseline: 1.2812x; 1.0240x over previous
import functools

import jax
import jax.numpy as jnp
from jax import lax
from jax.experimental import pallas as pl
from jax.experimental.pallas import tpu as pltpu
from jax.experimental.pallas import tpu_sc as plsc

NODE_N = {"tad": 10000, "atac_region": 50000, "gene": 20000, "protein": 20000, "gene_name": 20000}
D = 256
H = 128
RELS = [
    ("tad", "overlaps", "atac_region", 160000),
    ("atac_region", "rev_overlaps", "tad", 160000),
    ("tad", "overlaps", "gene", 80000),
    ("gene", "rev_overlaps", "tad", 80000),
    ("atac_region", "overlaps", "gene", 160000),
    ("gene", "rev_overlaps", "atac_region", 160000),
    ("protein", "coexpressed", "protein", 320000),
    ("protein", "tf_interacts", "gene", 80000),
    ("gene", "rev_tf_interacts", "protein", 80000),
    ("protein", "rev_associated", "gene", 80000),
    ("gene", "associated", "protein", 80000),
]
NAME_REL = ("protein", "is_named", "gene_name", 20000)
N_LAYERS = 4
NODE_TYPES = ["tad", "atac_region", "gene", "protein"]

NC = 2
NS = 16
WD = 4096
CHUNK = 512
ACC_ROWS = WD + 128
RPT = WD // NS
ZR = ACC_ROWS // NS


def _rk(r):
    return r[0] + "__" + r[1] + "__" + r[2]


def _ceil(a, b):
    return -(-a // b)


def _bin_edges(src, dst, n_dst, n_src):
    E = src.shape[0]
    NW = _ceil(n_dst, WD)
    w = dst // WD
    cnts = jnp.zeros((NW,), jnp.int32).at[w].add(1)
    nch = (cnts + (CHUNK - 1)) // CHUNK
    basech = jnp.cumsum(nch) - nch
    base_cnt = jnp.cumsum(cnts) - cnts
    order = jnp.argsort(w, stable=True)
    w_s = w[order]
    pos = basech[w_s] * CHUNK + (jnp.arange(E, dtype=jnp.int32) - base_cnt[w_s])
    E_pad = E + NW * CHUNK
    ar = jnp.arange(E_pad, dtype=jnp.int32)
    srcs_b = ((ar * 97) % n_src).at[pos].set(src[order])
    ldst_b = (WD + (ar % 128)).at[pos].set(dst[order] - w_s * WD)
    nch16 = jnp.zeros((16,), jnp.int32).at[:NW].set(nch)
    basech16 = jnp.zeros((16,), jnp.int32).at[:NW].set(basech)
    return srcs_b, ldst_b, nch16, basech16, NW


@functools.cache
def _segsum_call(n_src, E_pad, NW):
    out_rows = NW * WD
    mesh = plsc.VectorSubcoreMesh(core_axis_name="c", subcore_axis_name="s")

    @functools.partial(
        pl.kernel,
        out_type=jax.ShapeDtypeStruct((out_rows, H), jnp.float32),
        mesh=mesh,
        compiler_params=pltpu.CompilerParams(needs_layout_passes=False),
        scratch_types=[
            pltpu.VMEM((16,), jnp.int32),
            pltpu.VMEM((16,), jnp.int32),
            pltpu.VMEM((CHUNK,), jnp.int32),
            pltpu.VMEM((CHUNK,), jnp.int32),
            pltpu.VMEM((CHUNK, H), jnp.float32),
            pltpu.VMEM((64, H), jnp.float32),
            pltpu.VMEM_SHARED((ACC_ROWS, H), jnp.float32),
            pltpu.SemaphoreType.DMA,
        ],
    )
    def call(x_hbm, srcs_hbm, ldst_hbm, nch_hbm, basech_hbm, zeros_hbm, out_hbm,
             nch_v, basech_v, sidx_v, ldst_v, rows_v, zbuf_v, acc, sem):
        cid = lax.axis_index("c")
        sid = lax.axis_index("s")
        pltpu.sync_copy(zeros_hbm, zbuf_v)
        pltpu.sync_copy(nch_hbm, nch_v)
        pltpu.sync_copy(basech_hbm, basech_v)
        nch = nch_v[...]
        basech = basech_v[...]
        lanes = lax.broadcasted_iota(jnp.int32, (16,), 0)
        for w in range(NW):
            @pl.when(w % NC == cid)
            def _():
                z0 = sid * ZR
                for b in range(ZR // 64):
                    pltpu.sync_copy(zbuf_v, acc.at[pl.ds(z0 + b * 64, 64)])
                rem = ZR % 64
                if rem:
                    pltpu.sync_copy(zbuf_v.at[pl.ds(0, rem)],
                                    acc.at[pl.ds(z0 + (ZR // 64) * 64, rem)])
                plsc.subcore_barrier()
                nw = jnp.sum(jnp.where(lanes == w, nch, 0))
                b0 = jnp.sum(jnp.where(lanes == w, basech, 0))
                trips = (nw - sid + NS - 1) // NS

                def body(j, carry):
                    ch = sid + j * NS
                    off = (b0 + ch) * CHUNK
                    pltpu.sync_copy(srcs_hbm.at[pl.ds(off, CHUNK)], sidx_v)
                    pltpu.sync_copy(ldst_hbm.at[pl.ds(off, CHUNK)], ldst_v)
                    pltpu.async_copy(x_hbm.at[sidx_v], rows_v, sem).wait()
                    pltpu.sync_copy(rows_v, acc.at[ldst_v], add=True)
                    return carry

                lax.fori_loop(0, trips, body, 0)
                plsc.subcore_barrier()
                pltpu.sync_copy(acc.at[pl.ds(sid * RPT, RPT)],
                                out_hbm.at[pl.ds(w * WD + sid * RPT, RPT)])
                plsc.subcore_barrier()

    return call


@functools.cache
def _counts_call(E_pad, NW):
    out_rows = NW * WD
    mesh = plsc.VectorSubcoreMesh(core_axis_name="c", subcore_axis_name="s")

    @functools.partial(
        pl.kernel,
        out_type=jax.ShapeDtypeStruct((out_rows, H), jnp.float32),
        mesh=mesh,
        compiler_params=pltpu.CompilerParams(needs_layout_passes=False),
        scratch_types=[
            pltpu.VMEM((16,), jnp.int32),
            pltpu.VMEM((16,), jnp.int32),
            pltpu.VMEM((CHUNK,), jnp.int32),
            pltpu.VMEM((CHUNK, H), jnp.float32),
            pltpu.VMEM((64, H), jnp.float32),
            pltpu.VMEM_SHARED((ACC_ROWS, H), jnp.float32),
        ],
    )
    def call(ldst_hbm, nch_hbm, basech_hbm, ones_hbm, zeros_hbm, out_hbm,
             nch_v, basech_v, ldst_v, ones_v, zbuf_v, acc):
        cid = lax.axis_index("c")
        sid = lax.axis_index("s")
        pltpu.sync_copy(ones_hbm, ones_v)
        pltpu.sync_copy(zeros_hbm, zbuf_v)
        pltpu.sync_copy(nch_hbm, nch_v)
        pltpu.sync_copy(basech_hbm, basech_v)
        nch = nch_v[...]
        basech = basech_v[...]
        lanes = lax.broadcasted_iota(jnp.int32, (16,), 0)
        for w in range(NW):
            @pl.when(w % NC == cid)
            def _():
                z0 = sid * ZR
                for b in range(ZR // 64):
                    pltpu.sync_copy(zbuf_v, acc.at[pl.ds(z0 + b * 64, 64)])
                rem = ZR % 64
                if rem:
                    pltpu.sync_copy(zbuf_v.at[pl.ds(0, rem)],
                                    acc.at[pl.ds(z0 + (ZR // 64) * 64, rem)])
                plsc.subcore_barrier()
                nw = jnp.sum(jnp.where(lanes == w, nch, 0))
                b0 = jnp.sum(jnp.where(lanes == w, basech, 0))
                trips = (nw - sid + NS - 1) // NS

                def body(j, carry):
                    ch = sid + j * NS
                    off = (b0 + ch) * CHUNK
                    pltpu.sync_copy(ldst_hbm.at[pl.ds(off, CHUNK)], ldst_v)
                    pltpu.sync_copy(ones_v, acc.at[ldst_v], add=True)
                    return carry

                lax.fori_loop(0, trips, body, 0)
                plsc.subcore_barrier()
                pltpu.sync_copy(acc.at[pl.ds(sid * RPT, RPT)],
                                out_hbm.at[pl.ds(w * WD + sid * RPT, RPT)])
                plsc.subcore_barrier()

    return call


@functools.cache
def _conv_call(n, k, bm=400):
    grid = (n // bm,)

    def body(*refs):
        w_ref = refs[2 + 3 * k]
        b_ref = refs[3 + 3 * k]
        olo_ref = refs[4 + 3 * k]
        ohi_ref = refs[5 + 3 * k]
        x = jnp.concatenate([refs[0][...], refs[1][...]], axis=1)
        acc = jnp.dot(x, w_ref[0:D, :], preferred_element_type=jnp.float32)
        for i in range(k):
            seg = jnp.concatenate([refs[2 + 3 * i][...], refs[3 + 3 * i][...]],
                                  axis=1)
            cnt = refs[4 + 3 * i][...]
            scale = 1.0 / jnp.maximum(cnt[:, 0:1], 1.0)
            acc = acc + jnp.dot(seg * scale, w_ref[D * (i + 1):D * (i + 2), :],
                                preferred_element_type=jnp.float32)
        out = jnp.maximum(acc + b_ref[...], 0.0)
        olo_ref[...] = out[:, 0:H]
        ohi_ref[...] = out[:, H:D]

    in_specs = [pl.BlockSpec((bm, H), lambda i: (i, 0)),
                pl.BlockSpec((bm, H), lambda i: (i, 0))]
    for _ in range(k):
        in_specs.append(pl.BlockSpec((bm, H), lambda i: (i, 0)))
        in_specs.append(pl.BlockSpec((bm, H), lambda i: (i, 0)))
        in_specs.append(pl.BlockSpec((bm, H), lambda i: (i, 0)))
    in_specs.append(pl.BlockSpec((D * (k + 1), D), lambda i: (0, 0)))
    in_specs.append(pl.BlockSpec((1, D), lambda i: (0, 0)))

    return pl.pallas_call(
        body,
        grid=grid,
        in_specs=in_specs,
        out_specs=[pl.BlockSpec((bm, H), lambda i: (i, 0)),
                   pl.BlockSpec((bm, H), lambda i: (i, 0))],
        out_shape=[jax.ShapeDtypeStruct((n, H), jnp.float32),
                   jax.ShapeDtypeStruct((n, H), jnp.float32)],
    )


@functools.cache
def _head_call(n, bm=1000):
    grid = (n // bm,)

    def body(slo_ref, shi_ref, cnt_ref, w_ref, b_ref, o_ref):
        scale = 1.0 / jnp.maximum(cnt_ref[...][:, 0:1], 1.0)
        seg = jnp.concatenate([slo_ref[...], shi_ref[...]], axis=1)
        raw = jnp.dot(seg * scale, w_ref[...],
                      preferred_element_type=jnp.float32) + b_ref[...]
        lane = lax.broadcasted_iota(jnp.int32, (bm, 128), 1)
        o_ref[...] = jnp.where(lane == 1, jax.nn.sigmoid(raw), raw)

    return pl.pallas_call(
        body,
        grid=grid,
        in_specs=[
            pl.BlockSpec((bm, H), lambda i: (i, 0)),
            pl.BlockSpec((bm, H), lambda i: (i, 0)),
            pl.BlockSpec((bm, H), lambda i: (i, 0)),
            pl.BlockSpec((D, 128), lambda i: (0, 0)),
            pl.BlockSpec((1, 128), lambda i: (0, 0)),
        ],
        out_specs=pl.BlockSpec((bm, 128), lambda i: (i, 0)),
        out_shape=jax.ShapeDtypeStruct((n, 128), jnp.float32),
    )


def kernel(x_tad, x_atac_region, x_gene, x_protein, edge_index_tad__overlaps__atac_region, edge_index_atac_region__rev_overlaps__tad, edge_index_tad__overlaps__gene, edge_index_gene__rev_overlaps__tad, edge_index_atac_region__overlaps__gene, edge_index_gene__rev_overlaps__atac_region, edge_index_protein__coexpressed__protein, edge_index_protein__tf_interacts__gene, edge_index_gene__rev_tf_interacts__protein, edge_index_protein__rev_associated__gene, edge_index_gene__associated__protein, edge_index_protein__is_named__gene_name, params):
    kw = dict(locals())
    xs = {nt: kw["x_" + nt] for nt in NODE_TYPES}
    eis = {_rk(r): kw["edge_index_" + _rk(r)] for r in RELS + [NAME_REL]}

    zerosH = jnp.zeros((64, H), jnp.float32)
    onesH = jnp.ones((CHUNK, H), jnp.float32)

    bins, cnts = {}, {}
    for r in RELS + [NAME_REL]:
        k = _rk(r)
        ei = eis[k]
        srcs_b, ldst_b, nch16, basech16, NW = _bin_edges(
            ei[0], ei[1], NODE_N[r[2]], NODE_N[r[0]])
        bins[k] = (srcs_b, ldst_b, nch16, basech16, NW)
        cnts[k] = _counts_call(srcs_b.shape[0], NW)(
            ldst_b, nch16, basech16, onesH, zerosH)

    x = {t: (xs[t][:, 0:H], xs[t][:, H:D]) for t in NODE_TYPES}
    for l in range(N_LAYERS):
        lp = params["conv%d" % l]
        segs = {}
        for r in RELS:
            k = _rk(r)
            srcs_b, ldst_b, nch16, basech16, NW = bins[k]
            call = _segsum_call(NODE_N[r[0]], srcs_b.shape[0], NW)
            segs[k] = tuple(
                call(x[r[0]][h], srcs_b, ldst_b, nch16, basech16, zerosH)
                for h in range(2))
        new = {}
        for dst_t in NODE_TYPES:
            rels_t = [r for r in RELS if r[2] == dst_t]
            ks = [_rk(r) for r in rels_t]
            wr_sum = sum(lp[k]["W_r"] for k in ks)
            wcat = jnp.concatenate([wr_sum] + [lp[k]["W_l"] for k in ks], axis=0)
            bias = sum(lp[k]["b_l"] for k in ks).reshape(1, D)
            n = NODE_N[dst_t]
            args = [x[dst_t][0], x[dst_t][1]]
            for k in ks:
                args.append(segs[k][0])
                args.append(segs[k][1])
                args.append(cnts[k])
            new[dst_t] = tuple(_conv_call(n, len(ks))(*args, wcat, bias))
        x = new

    kn = _rk(NAME_REL)
    srcs_b, ldst_b, nch16, basech16, NW = bins[kn]
    call = _segsum_call(NODE_N["protein"], srcs_b.shape[0], NW)
    seg_n = tuple(
        call(x["protein"][h], srcs_b, ldst_b, nch16, basech16, zerosH)
        for h in range(2))
    p1, p2 = params["name_conv"], params["zero_conv"]
    w2 = jnp.zeros((D, 128), jnp.float32)
    w2 = w2.at[:, 0].set(p1["W_l"][:, 0]).at[:, 1].set(p2["W_l"][:, 0])
    b2 = jnp.zeros((1, 128), jnp.float32)
    b2 = b2.at[0, 0].set(p1["b_l"][0] - p1["W_r"][0, 0])
    b2 = b2.at[0, 1].set(p2["b_l"][0] - p2["W_r"][0, 0])
    outh = _head_call(NODE_N["gene_name"])(seg_n[0], seg_n[1], cnts[kn], w2, b2)
    return outh[:, :2]


# speedup vs baseline: 1.3421x; 1.0476x over previous
import functools

import jax
import jax.numpy as jnp
from jax import lax
from jax.experimental import pallas as pl
from jax.experimental.pallas import tpu as pltpu
from jax.experimental.pallas import tpu_sc as plsc

NODE_N = {"tad": 10000, "atac_region": 50000, "gene": 20000, "protein": 20000, "gene_name": 20000}
D = 256
H = 128
RELS = [
    ("tad", "overlaps", "atac_region", 160000),
    ("atac_region", "rev_overlaps", "tad", 160000),
    ("tad", "overlaps", "gene", 80000),
    ("gene", "rev_overlaps", "tad", 80000),
    ("atac_region", "overlaps", "gene", 160000),
    ("gene", "rev_overlaps", "atac_region", 160000),
    ("protein", "coexpressed", "protein", 320000),
    ("protein", "tf_interacts", "gene", 80000),
    ("gene", "rev_tf_interacts", "protein", 80000),
    ("protein", "rev_associated", "gene", 80000),
    ("gene", "associated", "protein", 80000),
]
NAME_REL = ("protein", "is_named", "gene_name", 20000)
N_LAYERS = 4
NODE_TYPES = ["tad", "atac_region", "gene", "protein"]

NC = 2
NS = 16
WD = 4096
CHUNK = 256
ACC_ROWS = WD + 128
RPT = WD // NS
ZR = ACC_ROWS // NS


def _rk(r):
    return r[0] + "__" + r[1] + "__" + r[2]


def _ceil(a, b):
    return -(-a // b)


def _bin_edges(src, dst, n_dst, n_src):
    E = src.shape[0]
    NW = _ceil(n_dst, WD)
    w = dst // WD
    cnts = jnp.zeros((NW,), jnp.int32).at[w].add(1)
    nch = (cnts + (CHUNK - 1)) // CHUNK
    basech = jnp.cumsum(nch) - nch
    base_cnt = jnp.cumsum(cnts) - cnts
    order = jnp.argsort(w, stable=True)
    w_s = w[order]
    pos = basech[w_s] * CHUNK + (jnp.arange(E, dtype=jnp.int32) - base_cnt[w_s])
    E_pad = E + NW * CHUNK
    ar = jnp.arange(E_pad, dtype=jnp.int32)
    srcs_b = ((ar * 97) % n_src).at[pos].set(src[order])
    ldst_b = (WD + (ar % 128)).at[pos].set(dst[order] - w_s * WD)
    nch16 = jnp.zeros((16,), jnp.int32).at[:NW].set(nch)
    basech16 = jnp.zeros((16,), jnp.int32).at[:NW].set(basech)
    return srcs_b, ldst_b, nch16, basech16, NW


@functools.cache
def _segsum_call(n_src, E_pad, NW):
    out_rows = NW * WD
    mesh = plsc.VectorSubcoreMesh(core_axis_name="c", subcore_axis_name="s")

    @functools.partial(
        pl.kernel,
        out_type=jax.ShapeDtypeStruct((out_rows, H), jnp.float32),
        mesh=mesh,
        compiler_params=pltpu.CompilerParams(needs_layout_passes=False),
        scratch_types=[
            pltpu.VMEM((16,), jnp.int32),
            pltpu.VMEM((16,), jnp.int32),
            pltpu.VMEM((CHUNK,), jnp.int32),
            pltpu.VMEM((CHUNK,), jnp.int32),
            pltpu.VMEM((CHUNK,), jnp.int32),
            pltpu.VMEM((CHUNK,), jnp.int32),
            pltpu.VMEM((CHUNK, H), jnp.float32),
            pltpu.VMEM((CHUNK, H), jnp.float32),
            pltpu.VMEM((64, H), jnp.float32),
            pltpu.VMEM_SHARED((ACC_ROWS, H), jnp.float32),
            pltpu.SemaphoreType.DMA,
            pltpu.SemaphoreType.DMA,
        ],
    )
    def call(x_hbm, srcs_hbm, ldst_hbm, nch_hbm, basech_hbm, zeros_hbm, out_hbm,
             nch_v, basech_v, sidx0_v, sidx1_v, ldst0_v, ldst1_v,
             rows0_v, rows1_v, zbuf_v, acc, sem0, sem1):
        cid = lax.axis_index("c")
        sid = lax.axis_index("s")
        pltpu.sync_copy(zeros_hbm, zbuf_v)
        pltpu.sync_copy(nch_hbm, nch_v)
        pltpu.sync_copy(basech_hbm, basech_v)
        nch = nch_v[...]
        basech = basech_v[...]
        lanes = lax.broadcasted_iota(jnp.int32, (16,), 0)

        def win_body(w, wcarry):
            @pl.when(w % NC == cid)
            def _():
                z0 = sid * ZR
                for b in range(ZR // 64):
                    pltpu.sync_copy(zbuf_v, acc.at[pl.ds(z0 + b * 64, 64)])
                rem = ZR % 64
                if rem:
                    pltpu.sync_copy(zbuf_v.at[pl.ds(0, rem)],
                                    acc.at[pl.ds(z0 + (ZR // 64) * 64, rem)])
                plsc.subcore_barrier()
                nw = jnp.sum(jnp.where(lanes == w, nch, 0))
                b0 = jnp.sum(jnp.where(lanes == w, basech, 0))
                trips = (nw - sid + NS - 1) // NS
                bufs = ((sidx0_v, ldst0_v, rows0_v, sem0),
                        (sidx1_v, ldst1_v, rows1_v, sem1))

                def off_of(j):
                    return (b0 + sid + j * NS) * CHUNK

                def prefetch(j, buf):
                    sidx, ldst, rows, sem = buf
                    off = off_of(j)
                    pltpu.sync_copy(srcs_hbm.at[pl.ds(off, CHUNK)], sidx)
                    pltpu.sync_copy(ldst_hbm.at[pl.ds(off, CHUNK)], ldst)
                    pltpu.async_copy(x_hbm.at[sidx], rows, sem)

                def consume(buf):
                    sidx, ldst, rows, sem = buf
                    pltpu.make_async_copy(x_hbm.at[sidx], rows, sem).wait()
                    pltpu.sync_copy(rows, acc.at[ldst], add=True)

                @pl.when(trips > 0)
                def _prologue():
                    prefetch(0, bufs[0])

                def pair_body(p, carry):
                    for sub in range(2):
                        j = p * 2 + sub

                        @pl.when(j < trips)
                        def _():
                            @pl.when(j + 1 < trips)
                            def _():
                                prefetch(j + 1, bufs[1 - sub])
                            consume(bufs[sub])
                    return carry

                lax.fori_loop(0, (trips + 1) // 2, pair_body, 0)
                plsc.subcore_barrier()
                out_off = pl.multiple_of(w * WD + sid * RPT, 8)
                pltpu.sync_copy(acc.at[pl.ds(sid * RPT, RPT)],
                                out_hbm.at[pl.ds(out_off, RPT)])
                plsc.subcore_barrier()
            return wcarry

        lax.fori_loop(0, NW, win_body, 0)

    return call


@functools.cache
def _counts_call(E_pad, NW):
    out_rows = NW * WD
    mesh = plsc.VectorSubcoreMesh(core_axis_name="c", subcore_axis_name="s")

    @functools.partial(
        pl.kernel,
        out_type=jax.ShapeDtypeStruct((out_rows, H), jnp.float32),
        mesh=mesh,
        compiler_params=pltpu.CompilerParams(needs_layout_passes=False),
        scratch_types=[
            pltpu.VMEM((16,), jnp.int32),
            pltpu.VMEM((16,), jnp.int32),
            pltpu.VMEM((CHUNK,), jnp.int32),
            pltpu.VMEM((CHUNK, H), jnp.float32),
            pltpu.VMEM((64, H), jnp.float32),
            pltpu.VMEM_SHARED((ACC_ROWS, H), jnp.float32),
        ],
    )
    def call(ldst_hbm, nch_hbm, basech_hbm, ones_hbm, zeros_hbm, out_hbm,
             nch_v, basech_v, ldst_v, ones_v, zbuf_v, acc):
        cid = lax.axis_index("c")
        sid = lax.axis_index("s")
        pltpu.sync_copy(ones_hbm, ones_v)
        pltpu.sync_copy(zeros_hbm, zbuf_v)
        pltpu.sync_copy(nch_hbm, nch_v)
        pltpu.sync_copy(basech_hbm, basech_v)
        nch = nch_v[...]
        basech = basech_v[...]
        lanes = lax.broadcasted_iota(jnp.int32, (16,), 0)

        def win_body(w, wcarry):
            @pl.when(w % NC == cid)
            def _():
                z0 = sid * ZR
                for b in range(ZR // 64):
                    pltpu.sync_copy(zbuf_v, acc.at[pl.ds(z0 + b * 64, 64)])
                rem = ZR % 64
                if rem:
                    pltpu.sync_copy(zbuf_v.at[pl.ds(0, rem)],
                                    acc.at[pl.ds(z0 + (ZR // 64) * 64, rem)])
                plsc.subcore_barrier()
                nw = jnp.sum(jnp.where(lanes == w, nch, 0))
                b0 = jnp.sum(jnp.where(lanes == w, basech, 0))
                trips = (nw - sid + NS - 1) // NS

                def body(j, carry):
                    ch = sid + j * NS
                    off = (b0 + ch) * CHUNK
                    pltpu.sync_copy(ldst_hbm.at[pl.ds(off, CHUNK)], ldst_v)
                    pltpu.sync_copy(ones_v, acc.at[ldst_v], add=True)
                    return carry

                lax.fori_loop(0, trips, body, 0)
                plsc.subcore_barrier()
                out_off = pl.multiple_of(w * WD + sid * RPT, 8)
                pltpu.sync_copy(acc.at[pl.ds(sid * RPT, RPT)],
                                out_hbm.at[pl.ds(out_off, RPT)])
                plsc.subcore_barrier()
            return wcarry

        lax.fori_loop(0, NW, win_body, 0)

    return call


@functools.cache
def _conv_call(n, k, bm=400):
    grid = (n // bm,)

    def body(*refs):
        w_ref = refs[2 + 3 * k]
        b_ref = refs[3 + 3 * k]
        olo_ref = refs[4 + 3 * k]
        ohi_ref = refs[5 + 3 * k]
        x = jnp.concatenate([refs[0][...], refs[1][...]], axis=1)
        acc = jnp.dot(x, w_ref[0:D, :], preferred_element_type=jnp.float32)
        for i in range(k):
            seg = jnp.concatenate([refs[2 + 3 * i][...], refs[3 + 3 * i][...]],
                                  axis=1)
            cnt = refs[4 + 3 * i][...]
            mean = seg / jnp.maximum(cnt[:, 0:1], 1.0)
            acc = acc + jnp.dot(mean, w_ref[D * (i + 1):D * (i + 2), :],
                                preferred_element_type=jnp.float32)
        out = jnp.maximum(acc + b_ref[...], 0.0)
        olo_ref[...] = out[:, 0:H]
        ohi_ref[...] = out[:, H:D]

    in_specs = [pl.BlockSpec((bm, H), lambda i: (i, 0)),
                pl.BlockSpec((bm, H), lambda i: (i, 0))]
    for _ in range(k):
        in_specs.append(pl.BlockSpec((bm, H), lambda i: (i, 0)))
        in_specs.append(pl.BlockSpec((bm, H), lambda i: (i, 0)))
        in_specs.append(pl.BlockSpec((bm, H), lambda i: (i, 0)))
    in_specs.append(pl.BlockSpec((D * (k + 1), D), lambda i: (0, 0)))
    in_specs.append(pl.BlockSpec((1, D), lambda i: (0, 0)))

    return pl.pallas_call(
        body,
        grid=grid,
        in_specs=in_specs,
        out_specs=[pl.BlockSpec((bm, H), lambda i: (i, 0)),
                   pl.BlockSpec((bm, H), lambda i: (i, 0))],
        out_shape=[jax.ShapeDtypeStruct((n, H), jnp.float32),
                   jax.ShapeDtypeStruct((n, H), jnp.float32)],
    )


@functools.cache
def _head_call(n, bm=1000):
    grid = (n // bm,)

    def body(slo_ref, shi_ref, cnt_ref, w_ref, b_ref, o_ref):
        seg = jnp.concatenate([slo_ref[...], shi_ref[...]], axis=1)
        mean = seg / jnp.maximum(cnt_ref[...][:, 0:1], 1.0)
        raw = jnp.dot(mean, w_ref[...],
                      preferred_element_type=jnp.float32) + b_ref[...]
        lane = lax.broadcasted_iota(jnp.int32, (bm, 128), 1)
        o_ref[...] = jnp.where(lane == 1, jax.nn.sigmoid(raw), raw)

    return pl.pallas_call(
        body,
        grid=grid,
        in_specs=[
            pl.BlockSpec((bm, H), lambda i: (i, 0)),
            pl.BlockSpec((bm, H), lambda i: (i, 0)),
            pl.BlockSpec((bm, H), lambda i: (i, 0)),
            pl.BlockSpec((D, 128), lambda i: (0, 0)),
            pl.BlockSpec((1, 128), lambda i: (0, 0)),
        ],
        out_specs=pl.BlockSpec((bm, 128), lambda i: (i, 0)),
        out_shape=jax.ShapeDtypeStruct((n, 128), jnp.float32),
    )


def kernel(x_tad, x_atac_region, x_gene, x_protein, edge_index_tad__overlaps__atac_region, edge_index_atac_region__rev_overlaps__tad, edge_index_tad__overlaps__gene, edge_index_gene__rev_overlaps__tad, edge_index_atac_region__overlaps__gene, edge_index_gene__rev_overlaps__atac_region, edge_index_protein__coexpressed__protein, edge_index_protein__tf_interacts__gene, edge_index_gene__rev_tf_interacts__protein, edge_index_protein__rev_associated__gene, edge_index_gene__associated__protein, edge_index_protein__is_named__gene_name, params):
    kw = dict(locals())
    xs = {nt: kw["x_" + nt] for nt in NODE_TYPES}
    eis = {_rk(r): kw["edge_index_" + _rk(r)] for r in RELS + [NAME_REL]}

    zerosH = jnp.zeros((64, H), jnp.float32)
    onesH = jnp.ones((CHUNK, H), jnp.float32)

    bins, cnts = {}, {}
    for r in RELS + [NAME_REL]:
        k = _rk(r)
        ei = eis[k]
        srcs_b, ldst_b, nch16, basech16, NW = _bin_edges(
            ei[0], ei[1], NODE_N[r[2]], NODE_N[r[0]])
        bins[k] = (srcs_b, ldst_b, nch16, basech16, NW)
        cnts[k] = _counts_call(srcs_b.shape[0], NW)(
            ldst_b, nch16, basech16, onesH, zerosH)

    x = {t: (xs[t][:, 0:H], xs[t][:, H:D]) for t in NODE_TYPES}
    for l in range(N_LAYERS):
        lp = params["conv%d" % l]
        segs = {}
        for r in RELS:
            k = _rk(r)
            srcs_b, ldst_b, nch16, basech16, NW = bins[k]
            call = _segsum_call(NODE_N[r[0]], srcs_b.shape[0], NW)
            segs[k] = tuple(
                call(x[r[0]][h], srcs_b, ldst_b, nch16, basech16, zerosH)
                for h in range(2))
        new = {}
        for dst_t in NODE_TYPES:
            rels_t = [r for r in RELS if r[2] == dst_t]
            ks = [_rk(r) for r in rels_t]
            wr_sum = sum(lp[k]["W_r"] for k in ks)
            wcat = jnp.concatenate([wr_sum] + [lp[k]["W_l"] for k in ks], axis=0)
            bias = sum(lp[k]["b_l"] for k in ks).reshape(1, D)
            n = NODE_N[dst_t]
            args = [x[dst_t][0], x[dst_t][1]]
            for k in ks:
                args.append(segs[k][0])
                args.append(segs[k][1])
                args.append(cnts[k])
            new[dst_t] = tuple(_conv_call(n, len(ks))(*args, wcat, bias))
        x = new

    kn = _rk(NAME_REL)
    srcs_b, ldst_b, nch16, basech16, NW = bins[kn]
    call = _segsum_call(NODE_N["protein"], srcs_b.shape[0], NW)
    seg_n = tuple(
        call(x["protein"][h], srcs_b, ldst_b, nch16, basech16, zerosH)
        for h in range(2))
    p1, p2 = params["name_conv"], params["zero_conv"]
    w2 = jnp.zeros((D, 128), jnp.float32)
    w2 = w2.at[:, 0].set(p1["W_l"][:, 0]).at[:, 1].set(p2["W_l"][:, 0])
    b2 = jnp.zeros((1, 128), jnp.float32)
    b2 = b2.at[0, 0].set(p1["b_l"][0] - p1["W_r"][0, 0])
    b2 = b2.at[0, 1].set(p2["b_l"][0] - p2["W_r"][0, 0])
    outh = _head_call(NODE_N["gene_name"])(seg_n[0], seg_n[1], cnts[kn], w2, b2)
    return outh[:, :2]


# speedup vs baseline: 2.4502x; 1.8256x over previous
import functools

import jax
import jax.numpy as jnp
from jax import lax
from jax.experimental import pallas as pl
from jax.experimental.pallas import tpu as pltpu
from jax.experimental.pallas import tpu_sc as plsc

NODE_N = {"tad": 10000, "atac_region": 50000, "gene": 20000, "protein": 20000, "gene_name": 20000}
D = 256
H = 128
RELS = [
    ("tad", "overlaps", "atac_region", 160000),
    ("atac_region", "rev_overlaps", "tad", 160000),
    ("tad", "overlaps", "gene", 80000),
    ("gene", "rev_overlaps", "tad", 80000),
    ("atac_region", "overlaps", "gene", 160000),
    ("gene", "rev_overlaps", "atac_region", 160000),
    ("protein", "coexpressed", "protein", 320000),
    ("protein", "tf_interacts", "gene", 80000),
    ("gene", "rev_tf_interacts", "protein", 80000),
    ("protein", "rev_associated", "gene", 80000),
    ("gene", "associated", "protein", 80000),
]
NAME_REL = ("protein", "is_named", "gene_name", 20000)
N_LAYERS = 4
NODE_TYPES = ["tad", "atac_region", "gene", "protein"]

NC = 2
NS = 16
WD = 4096
CHUNK = 256
ACC_ROWS = WD + 128
RPT = WD // NS
ZR = ACC_ROWS // NS


def _rk(r):
    return r[0] + "__" + r[1] + "__" + r[2]


def _ceil(a, b):
    return -(-a // b)


def _bin_edges(src, dst, n_dst, n_src):
    E = src.shape[0]
    NW = _ceil(n_dst, WD)
    w = dst // WD
    cnts = jnp.zeros((NW,), jnp.int32).at[w].add(1)
    nch = (cnts + (CHUNK - 1)) // CHUNK
    basech = jnp.cumsum(nch) - nch
    base_cnt = jnp.cumsum(cnts) - cnts
    order = jnp.argsort(w, stable=True)
    E_pad = E + NW * CHUNK
    p = jnp.arange(E_pad, dtype=jnp.int32)
    base_el = basech * CHUNK
    ws = jnp.sum((p[:, None] >= base_el[None, :]).astype(jnp.int32), axis=1) - 1
    rank = p - base_el[ws]
    valid = rank < cnts[ws]
    e_idx = jnp.where(valid, base_cnt[ws] + jnp.minimum(rank, cnts[ws] - 1), 0)
    esrc = src[order[e_idx]]
    eldst = dst[order[e_idx]] - ws * WD
    srcs_b = jnp.where(valid, esrc, (p * 97) % n_src)
    ldst_b = jnp.where(valid, eldst, WD + (p % 128))
    nch16 = jnp.zeros((16,), jnp.int32).at[:NW].set(nch)
    basech16 = jnp.zeros((16,), jnp.int32).at[:NW].set(basech)
    return srcs_b, ldst_b, nch16, basech16, NW


@functools.cache
def _segsum_call(n_src, E_pad, NW):
    out_rows = NW * WD
    mesh = plsc.VectorSubcoreMesh(core_axis_name="c", subcore_axis_name="s")

    @functools.partial(
        pl.kernel,
        out_type=jax.ShapeDtypeStruct((out_rows, H), jnp.float32),
        mesh=mesh,
        compiler_params=pltpu.CompilerParams(needs_layout_passes=False),
        scratch_types=[
            pltpu.VMEM((16,), jnp.int32),
            pltpu.VMEM((16,), jnp.int32),
            pltpu.VMEM((CHUNK,), jnp.int32),
            pltpu.VMEM((CHUNK,), jnp.int32),
            pltpu.VMEM((CHUNK,), jnp.int32),
            pltpu.VMEM((CHUNK,), jnp.int32),
            pltpu.VMEM((CHUNK, H), jnp.float32),
            pltpu.VMEM((CHUNK, H), jnp.float32),
            pltpu.VMEM((64, H), jnp.float32),
            pltpu.VMEM_SHARED((ACC_ROWS, H), jnp.float32),
            pltpu.SemaphoreType.DMA,
            pltpu.SemaphoreType.DMA,
        ],
    )
    def call(x_hbm, srcs_hbm, ldst_hbm, nch_hbm, basech_hbm, zeros_hbm, out_hbm,
             nch_v, basech_v, sidx0_v, sidx1_v, ldst0_v, ldst1_v,
             rows0_v, rows1_v, zbuf_v, acc, sem0, sem1):
        cid = lax.axis_index("c")
        sid = lax.axis_index("s")
        pltpu.sync_copy(zeros_hbm, zbuf_v)
        pltpu.sync_copy(nch_hbm, nch_v)
        pltpu.sync_copy(basech_hbm, basech_v)
        nch = nch_v[...]
        basech = basech_v[...]
        lanes = lax.broadcasted_iota(jnp.int32, (16,), 0)

        def win_body(w, wcarry):
            @pl.when(w % NC == cid)
            def _():
                z0 = sid * ZR
                for b in range(ZR // 64):
                    pltpu.sync_copy(zbuf_v, acc.at[pl.ds(z0 + b * 64, 64)])
                rem = ZR % 64
                if rem:
                    pltpu.sync_copy(zbuf_v.at[pl.ds(0, rem)],
                                    acc.at[pl.ds(z0 + (ZR // 64) * 64, rem)])
                plsc.subcore_barrier()
                nw = jnp.sum(jnp.where(lanes == w, nch, 0))
                b0 = jnp.sum(jnp.where(lanes == w, basech, 0))
                trips = (nw - sid + NS - 1) // NS
                bufs = ((sidx0_v, ldst0_v, rows0_v, sem0),
                        (sidx1_v, ldst1_v, rows1_v, sem1))

                def off_of(j):
                    return (b0 + sid + j * NS) * CHUNK

                def prefetch(j, buf):
                    sidx, ldst, rows, sem = buf
                    off = off_of(j)
                    pltpu.sync_copy(srcs_hbm.at[pl.ds(off, CHUNK)], sidx)
                    pltpu.sync_copy(ldst_hbm.at[pl.ds(off, CHUNK)], ldst)
                    pltpu.async_copy(x_hbm.at[sidx], rows, sem)

                def consume(buf):
                    sidx, ldst, rows, sem = buf
                    pltpu.make_async_copy(x_hbm.at[sidx], rows, sem).wait()
                    pltpu.sync_copy(rows, acc.at[ldst], add=True)

                @pl.when(trips > 0)
                def _prologue():
                    prefetch(0, bufs[0])

                def pair_body(p, carry):
                    for sub in range(2):
                        j = p * 2 + sub

                        @pl.when(j < trips)
                        def _():
                            @pl.when(j + 1 < trips)
                            def _():
                                prefetch(j + 1, bufs[1 - sub])
                            consume(bufs[sub])
                    return carry

                lax.fori_loop(0, (trips + 1) // 2, pair_body, 0)
                plsc.subcore_barrier()
                out_off = pl.multiple_of(w * WD + sid * RPT, 8)
                pltpu.sync_copy(acc.at[pl.ds(sid * RPT, RPT)],
                                out_hbm.at[pl.ds(out_off, RPT)])
                plsc.subcore_barrier()
            return wcarry

        lax.fori_loop(0, NW, win_body, 0)

    return call


@functools.cache
def _counts_call(E_pad, NW):
    out_rows = NW * WD
    mesh = plsc.VectorSubcoreMesh(core_axis_name="c", subcore_axis_name="s")

    @functools.partial(
        pl.kernel,
        out_type=jax.ShapeDtypeStruct((out_rows, H), jnp.float32),
        mesh=mesh,
        compiler_params=pltpu.CompilerParams(needs_layout_passes=False),
        scratch_types=[
            pltpu.VMEM((16,), jnp.int32),
            pltpu.VMEM((16,), jnp.int32),
            pltpu.VMEM((CHUNK,), jnp.int32),
            pltpu.VMEM((CHUNK, H), jnp.float32),
            pltpu.VMEM((64, H), jnp.float32),
            pltpu.VMEM_SHARED((ACC_ROWS, H), jnp.float32),
        ],
    )
    def call(ldst_hbm, nch_hbm, basech_hbm, ones_hbm, zeros_hbm, out_hbm,
             nch_v, basech_v, ldst_v, ones_v, zbuf_v, acc):
        cid = lax.axis_index("c")
        sid = lax.axis_index("s")
        pltpu.sync_copy(ones_hbm, ones_v)
        pltpu.sync_copy(zeros_hbm, zbuf_v)
        pltpu.sync_copy(nch_hbm, nch_v)
        pltpu.sync_copy(basech_hbm, basech_v)
        nch = nch_v[...]
        basech = basech_v[...]
        lanes = lax.broadcasted_iota(jnp.int32, (16,), 0)

        def win_body(w, wcarry):
            @pl.when(w % NC == cid)
            def _():
                z0 = sid * ZR
                for b in range(ZR // 64):
                    pltpu.sync_copy(zbuf_v, acc.at[pl.ds(z0 + b * 64, 64)])
                rem = ZR % 64
                if rem:
                    pltpu.sync_copy(zbuf_v.at[pl.ds(0, rem)],
                                    acc.at[pl.ds(z0 + (ZR // 64) * 64, rem)])
                plsc.subcore_barrier()
                nw = jnp.sum(jnp.where(lanes == w, nch, 0))
                b0 = jnp.sum(jnp.where(lanes == w, basech, 0))
                trips = (nw - sid + NS - 1) // NS

                def body(j, carry):
                    ch = sid + j * NS
                    off = (b0 + ch) * CHUNK
                    pltpu.sync_copy(ldst_hbm.at[pl.ds(off, CHUNK)], ldst_v)
                    pltpu.sync_copy(ones_v, acc.at[ldst_v], add=True)
                    return carry

                lax.fori_loop(0, trips, body, 0)
                plsc.subcore_barrier()
                out_off = pl.multiple_of(w * WD + sid * RPT, 8)
                pltpu.sync_copy(acc.at[pl.ds(sid * RPT, RPT)],
                                out_hbm.at[pl.ds(out_off, RPT)])
                plsc.subcore_barrier()
            return wcarry

        lax.fori_loop(0, NW, win_body, 0)

    return call


@functools.cache
def _conv_call(n, k, bm=400):
    grid = (n // bm,)

    def body(*refs):
        w_ref = refs[2 + 3 * k]
        b_ref = refs[3 + 3 * k]
        olo_ref = refs[4 + 3 * k]
        ohi_ref = refs[5 + 3 * k]
        x = jnp.concatenate([refs[0][...], refs[1][...]], axis=1)
        acc = jnp.dot(x, w_ref[0:D, :], preferred_element_type=jnp.float32)
        for i in range(k):
            seg = jnp.concatenate([refs[2 + 3 * i][...], refs[3 + 3 * i][...]],
                                  axis=1)
            cnt = refs[4 + 3 * i][...]
            mean = seg / jnp.maximum(cnt[:, 0:1], 1.0)
            acc = acc + jnp.dot(mean, w_ref[D * (i + 1):D * (i + 2), :],
                                preferred_element_type=jnp.float32)
        out = jnp.maximum(acc + b_ref[...], 0.0)
        olo_ref[...] = out[:, 0:H]
        ohi_ref[...] = out[:, H:D]

    in_specs = [pl.BlockSpec((bm, H), lambda i: (i, 0)),
                pl.BlockSpec((bm, H), lambda i: (i, 0))]
    for _ in range(k):
        in_specs.append(pl.BlockSpec((bm, H), lambda i: (i, 0)))
        in_specs.append(pl.BlockSpec((bm, H), lambda i: (i, 0)))
        in_specs.append(pl.BlockSpec((bm, H), lambda i: (i, 0)))
    in_specs.append(pl.BlockSpec((D * (k + 1), D), lambda i: (0, 0)))
    in_specs.append(pl.BlockSpec((1, D), lambda i: (0, 0)))

    return pl.pallas_call(
        body,
        grid=grid,
        in_specs=in_specs,
        out_specs=[pl.BlockSpec((bm, H), lambda i: (i, 0)),
                   pl.BlockSpec((bm, H), lambda i: (i, 0))],
        out_shape=[jax.ShapeDtypeStruct((n, H), jnp.float32),
                   jax.ShapeDtypeStruct((n, H), jnp.float32)],
    )


@functools.cache
def _head_call(n, bm=1000):
    grid = (n // bm,)

    def body(slo_ref, shi_ref, cnt_ref, w_ref, b_ref, o_ref):
        seg = jnp.concatenate([slo_ref[...], shi_ref[...]], axis=1)
        mean = seg / jnp.maximum(cnt_ref[...][:, 0:1], 1.0)
        raw = jnp.dot(mean, w_ref[...],
                      preferred_element_type=jnp.float32) + b_ref[...]
        lane = lax.broadcasted_iota(jnp.int32, (bm, 128), 1)
        o_ref[...] = jnp.where(lane == 1, jax.nn.sigmoid(raw), raw)

    return pl.pallas_call(
        body,
        grid=grid,
        in_specs=[
            pl.BlockSpec((bm, H), lambda i: (i, 0)),
            pl.BlockSpec((bm, H), lambda i: (i, 0)),
            pl.BlockSpec((bm, H), lambda i: (i, 0)),
            pl.BlockSpec((D, 128), lambda i: (0, 0)),
            pl.BlockSpec((1, 128), lambda i: (0, 0)),
        ],
        out_specs=pl.BlockSpec((bm, 128), lambda i: (i, 0)),
        out_shape=jax.ShapeDtypeStruct((n, 128), jnp.float32),
    )


def kernel(x_tad, x_atac_region, x_gene, x_protein, edge_index_tad__overlaps__atac_region, edge_index_atac_region__rev_overlaps__tad, edge_index_tad__overlaps__gene, edge_index_gene__rev_overlaps__tad, edge_index_atac_region__overlaps__gene, edge_index_gene__rev_overlaps__atac_region, edge_index_protein__coexpressed__protein, edge_index_protein__tf_interacts__gene, edge_index_gene__rev_tf_interacts__protein, edge_index_protein__rev_associated__gene, edge_index_gene__associated__protein, edge_index_protein__is_named__gene_name, params):
    kw = dict(locals())
    xs = {nt: kw["x_" + nt] for nt in NODE_TYPES}
    eis = {_rk(r): kw["edge_index_" + _rk(r)] for r in RELS + [NAME_REL]}

    zerosH = jnp.zeros((64, H), jnp.float32)
    onesH = jnp.ones((CHUNK, H), jnp.float32)

    bins, cnts = {}, {}
    for r in RELS + [NAME_REL]:
        k = _rk(r)
        ei = eis[k]
        srcs_b, ldst_b, nch16, basech16, NW = _bin_edges(
            ei[0], ei[1], NODE_N[r[2]], NODE_N[r[0]])
        bins[k] = (srcs_b, ldst_b, nch16, basech16, NW)
        cnts[k] = _counts_call(srcs_b.shape[0], NW)(
            ldst_b, nch16, basech16, onesH, zerosH)

    x = {t: (xs[t][:, 0:H], xs[t][:, H:D]) for t in NODE_TYPES}
    for l in range(N_LAYERS):
        lp = params["conv%d" % l]
        segs = {}
        for r in RELS:
            k = _rk(r)
            srcs_b, ldst_b, nch16, basech16, NW = bins[k]
            call = _segsum_call(NODE_N[r[0]], srcs_b.shape[0], NW)
            segs[k] = tuple(
                call(x[r[0]][h], srcs_b, ldst_b, nch16, basech16, zerosH)
                for h in range(2))
        new = {}
        for dst_t in NODE_TYPES:
            rels_t = [r for r in RELS if r[2] == dst_t]
            ks = [_rk(r) for r in rels_t]
            wr_sum = sum(lp[k]["W_r"] for k in ks)
            wcat = jnp.concatenate([wr_sum] + [lp[k]["W_l"] for k in ks], axis=0)
            bias = sum(lp[k]["b_l"] for k in ks).reshape(1, D)
            n = NODE_N[dst_t]
            args = [x[dst_t][0], x[dst_t][1]]
            for k in ks:
                args.append(segs[k][0])
                args.append(segs[k][1])
                args.append(cnts[k])
            new[dst_t] = tuple(_conv_call(n, len(ks))(*args, wcat, bias))
        x = new

    kn = _rk(NAME_REL)
    srcs_b, ldst_b, nch16, basech16, NW = bins[kn]
    call = _segsum_call(NODE_N["protein"], srcs_b.shape[0], NW)
    seg_n = tuple(
        call(x["protein"][h], srcs_b, ldst_b, nch16, basech16, zerosH)
        for h in range(2))
    p1, p2 = params["name_conv"], params["zero_conv"]
    w2 = jnp.zeros((D, 128), jnp.float32)
    w2 = w2.at[:, 0].set(p1["W_l"][:, 0]).at[:, 1].set(p2["W_l"][:, 0])
    b2 = jnp.zeros((1, 128), jnp.float32)
    b2 = b2.at[0, 0].set(p1["b_l"][0] - p1["W_r"][0, 0])
    b2 = b2.at[0, 1].set(p2["b_l"][0] - p2["W_r"][0, 0])
    outh = _head_call(NODE_N["gene_name"])(seg_n[0], seg_n[1], cnts[kn], w2, b2)
    return outh[:, :2]


# speedup vs baseline: 2.5120x; 1.0252x over previous
import functools

import jax
import jax.numpy as jnp
from jax import lax
from jax.experimental import pallas as pl
from jax.experimental.pallas import tpu as pltpu
from jax.experimental.pallas import tpu_sc as plsc

NODE_N = {"tad": 10000, "atac_region": 50000, "gene": 20000, "protein": 20000, "gene_name": 20000}
D = 256
H = 128
RELS = [
    ("tad", "overlaps", "atac_region", 160000),
    ("atac_region", "rev_overlaps", "tad", 160000),
    ("tad", "overlaps", "gene", 80000),
    ("gene", "rev_overlaps", "tad", 80000),
    ("atac_region", "overlaps", "gene", 160000),
    ("gene", "rev_overlaps", "atac_region", 160000),
    ("protein", "coexpressed", "protein", 320000),
    ("protein", "tf_interacts", "gene", 80000),
    ("gene", "rev_tf_interacts", "protein", 80000),
    ("protein", "rev_associated", "gene", 80000),
    ("gene", "associated", "protein", 80000),
]
NAME_REL = ("protein", "is_named", "gene_name", 20000)
N_LAYERS = 4
NODE_TYPES = ["tad", "atac_region", "gene", "protein"]

NC = 2
NS = 16
WD = 4096
CHUNK = 256
ACC_ROWS = WD + 128
RPT = WD // NS
ZR = ACC_ROWS // NS


def _rk(r):
    return r[0] + "__" + r[1] + "__" + r[2]


def _ceil(a, b):
    return -(-a // b)


def _bin_edges(src, dst, n_dst, n_src):
    E = src.shape[0]
    NW = _ceil(n_dst, WD)
    w = dst // WD
    cnts = jnp.zeros((NW,), jnp.int32).at[w].add(1)
    nch = (cnts + (CHUNK - 1)) // CHUNK
    basech = jnp.cumsum(nch) - nch
    base_cnt = jnp.cumsum(cnts) - cnts
    order = jnp.argsort(w, stable=True)
    E_pad = E + NW * CHUNK
    p = jnp.arange(E_pad, dtype=jnp.int32)
    base_el = basech * CHUNK
    ws = jnp.sum((p[:, None] >= base_el[None, :]).astype(jnp.int32), axis=1) - 1
    rank = p - base_el[ws]
    valid = rank < cnts[ws]
    e_idx = jnp.where(valid, base_cnt[ws] + jnp.minimum(rank, cnts[ws] - 1), 0)
    esrc = src[order[e_idx]]
    eldst = dst[order[e_idx]] - ws * WD
    srcs_b = jnp.where(valid, esrc, (p * 97) % n_src)
    ldst_b = jnp.where(valid, eldst, WD + (p % 128))
    nch16 = jnp.zeros((16,), jnp.int32).at[:NW].set(nch)
    basech16 = jnp.zeros((16,), jnp.int32).at[:NW].set(basech)
    return srcs_b, ldst_b, nch16, basech16, NW


@functools.cache
def _segsum_call(n_src, E_pad, NW):
    out_rows = NW * WD
    mesh = plsc.VectorSubcoreMesh(core_axis_name="c", subcore_axis_name="s")

    @functools.partial(
        pl.kernel,
        out_type=jax.ShapeDtypeStruct((out_rows, H), jnp.float32),
        mesh=mesh,
        compiler_params=pltpu.CompilerParams(needs_layout_passes=False),
        scratch_types=[
            pltpu.VMEM((16,), jnp.int32),
            pltpu.VMEM((16,), jnp.int32),
            pltpu.VMEM((CHUNK,), jnp.int32),
            pltpu.VMEM((CHUNK,), jnp.int32),
            pltpu.VMEM((CHUNK,), jnp.int32),
            pltpu.VMEM((CHUNK,), jnp.int32),
            pltpu.VMEM((CHUNK, H), jnp.float32),
            pltpu.VMEM((CHUNK, H), jnp.float32),
            pltpu.VMEM((64, H), jnp.float32),
            pltpu.VMEM_SHARED((ACC_ROWS, H), jnp.float32),
            pltpu.SemaphoreType.DMA,
            pltpu.SemaphoreType.DMA,
        ],
    )
    def call(x_hbm, srcs_hbm, ldst_hbm, nch_hbm, basech_hbm, zeros_hbm, out_hbm,
             nch_v, basech_v, sidx0_v, sidx1_v, ldst0_v, ldst1_v,
             rows0_v, rows1_v, zbuf_v, acc, sem0, sem1):
        cid = lax.axis_index("c")
        sid = lax.axis_index("s")
        pltpu.sync_copy(zeros_hbm, zbuf_v)
        pltpu.sync_copy(nch_hbm, nch_v)
        pltpu.sync_copy(basech_hbm, basech_v)
        nch = nch_v[...]
        basech = basech_v[...]
        lanes = lax.broadcasted_iota(jnp.int32, (16,), 0)

        def win_body(w, wcarry):
            @pl.when(w % NC == cid)
            def _():
                z0 = sid * ZR
                for b in range(ZR // 64):
                    pltpu.sync_copy(zbuf_v, acc.at[pl.ds(z0 + b * 64, 64)])
                rem = ZR % 64
                if rem:
                    pltpu.sync_copy(zbuf_v.at[pl.ds(0, rem)],
                                    acc.at[pl.ds(z0 + (ZR // 64) * 64, rem)])
                plsc.subcore_barrier()
                nw = jnp.sum(jnp.where(lanes == w, nch, 0))
                b0 = jnp.sum(jnp.where(lanes == w, basech, 0))
                trips = (nw - sid + NS - 1) // NS
                bufs = ((sidx0_v, ldst0_v, rows0_v, sem0),
                        (sidx1_v, ldst1_v, rows1_v, sem1))

                def off_of(j):
                    return (b0 + sid + j * NS) * CHUNK

                def prefetch(j, buf):
                    sidx, ldst, rows, sem = buf
                    off = off_of(j)
                    pltpu.sync_copy(srcs_hbm.at[pl.ds(off, CHUNK)], sidx)
                    pltpu.sync_copy(ldst_hbm.at[pl.ds(off, CHUNK)], ldst)
                    pltpu.async_copy(x_hbm.at[sidx], rows, sem)

                def consume(buf):
                    sidx, ldst, rows, sem = buf
                    pltpu.make_async_copy(x_hbm.at[sidx], rows, sem).wait()
                    pltpu.sync_copy(rows, acc.at[ldst], add=True)

                @pl.when(trips > 0)
                def _prologue():
                    prefetch(0, bufs[0])

                def pair_body(p, carry):
                    for sub in range(2):
                        j = p * 2 + sub

                        @pl.when(j < trips)
                        def _():
                            @pl.when(j + 1 < trips)
                            def _():
                                prefetch(j + 1, bufs[1 - sub])
                            consume(bufs[sub])
                    return carry

                lax.fori_loop(0, (trips + 1) // 2, pair_body, 0)
                plsc.subcore_barrier()
                out_off = pl.multiple_of(w * WD + sid * RPT, 8)
                pltpu.sync_copy(acc.at[pl.ds(sid * RPT, RPT)],
                                out_hbm.at[pl.ds(out_off, RPT)])
                plsc.subcore_barrier()
            return wcarry

        lax.fori_loop(0, NW, win_body, 0)

    return call


@functools.cache
def _counts_call(E_pad, NW):
    out_rows = NW * WD
    mesh = plsc.VectorSubcoreMesh(core_axis_name="c", subcore_axis_name="s")

    @functools.partial(
        pl.kernel,
        out_type=jax.ShapeDtypeStruct((out_rows, 16), jnp.float32),
        mesh=mesh,
        compiler_params=pltpu.CompilerParams(needs_layout_passes=False),
        scratch_types=[
            pltpu.VMEM((16,), jnp.int32),
            pltpu.VMEM((16,), jnp.int32),
            pltpu.VMEM((CHUNK,), jnp.int32),
            pltpu.VMEM((CHUNK, 16), jnp.float32),
            pltpu.VMEM((64, 16), jnp.float32),
            pltpu.VMEM_SHARED((ACC_ROWS, 16), jnp.float32),
        ],
    )
    def call(ldst_hbm, nch_hbm, basech_hbm, ones_hbm, zeros_hbm, out_hbm,
             nch_v, basech_v, ldst_v, ones_v, zbuf_v, acc):
        cid = lax.axis_index("c")
        sid = lax.axis_index("s")
        pltpu.sync_copy(ones_hbm, ones_v)
        pltpu.sync_copy(zeros_hbm, zbuf_v)
        pltpu.sync_copy(nch_hbm, nch_v)
        pltpu.sync_copy(basech_hbm, basech_v)
        nch = nch_v[...]
        basech = basech_v[...]
        lanes = lax.broadcasted_iota(jnp.int32, (16,), 0)

        def win_body(w, wcarry):
            @pl.when(w % NC == cid)
            def _():
                z0 = sid * ZR
                for b in range(ZR // 64):
                    pltpu.sync_copy(zbuf_v, acc.at[pl.ds(z0 + b * 64, 64)])
                rem = ZR % 64
                if rem:
                    pltpu.sync_copy(zbuf_v.at[pl.ds(0, rem)],
                                    acc.at[pl.ds(z0 + (ZR // 64) * 64, rem)])
                plsc.subcore_barrier()
                nw = jnp.sum(jnp.where(lanes == w, nch, 0))
                b0 = jnp.sum(jnp.where(lanes == w, basech, 0))
                trips = (nw - sid + NS - 1) // NS

                def body(j, carry):
                    ch = sid + j * NS
                    off = (b0 + ch) * CHUNK
                    pltpu.sync_copy(ldst_hbm.at[pl.ds(off, CHUNK)], ldst_v)
                    pltpu.sync_copy(ones_v, acc.at[ldst_v], add=True)
                    return carry

                lax.fori_loop(0, trips, body, 0)
                plsc.subcore_barrier()
                out_off = pl.multiple_of(w * WD + sid * RPT, 8)
                pltpu.sync_copy(acc.at[pl.ds(sid * RPT, RPT)],
                                out_hbm.at[pl.ds(out_off, RPT)])
                plsc.subcore_barrier()
            return wcarry

        lax.fori_loop(0, NW, win_body, 0)

    return call


@functools.cache
def _conv_call(n, k, bm=400):
    grid = (n // bm,)

    def body(*refs):
        w_ref = refs[2 + 3 * k]
        b_ref = refs[3 + 3 * k]
        olo_ref = refs[4 + 3 * k]
        ohi_ref = refs[5 + 3 * k]
        x = jnp.concatenate([refs[0][...], refs[1][...]], axis=1)
        acc = jnp.dot(x, w_ref[0:D, :], preferred_element_type=jnp.float32)
        for i in range(k):
            seg = jnp.concatenate([refs[2 + 3 * i][...], refs[3 + 3 * i][...]],
                                  axis=1)
            cnt = refs[4 + 3 * i][...]
            mean = seg / jnp.maximum(cnt[:, 0:1], 1.0)
            acc = acc + jnp.dot(mean, w_ref[D * (i + 1):D * (i + 2), :],
                                preferred_element_type=jnp.float32)
        out = jnp.maximum(acc + b_ref[...], 0.0)
        olo_ref[...] = out[:, 0:H]
        ohi_ref[...] = out[:, H:D]

    in_specs = [pl.BlockSpec((bm, H), lambda i: (i, 0)),
                pl.BlockSpec((bm, H), lambda i: (i, 0))]
    for _ in range(k):
        in_specs.append(pl.BlockSpec((bm, H), lambda i: (i, 0)))
        in_specs.append(pl.BlockSpec((bm, H), lambda i: (i, 0)))
        in_specs.append(pl.BlockSpec((bm, 16), lambda i: (i, 0)))
    in_specs.append(pl.BlockSpec((D * (k + 1), D), lambda i: (0, 0)))
    in_specs.append(pl.BlockSpec((1, D), lambda i: (0, 0)))

    return pl.pallas_call(
        body,
        grid=grid,
        in_specs=in_specs,
        out_specs=[pl.BlockSpec((bm, H), lambda i: (i, 0)),
                   pl.BlockSpec((bm, H), lambda i: (i, 0))],
        out_shape=[jax.ShapeDtypeStruct((n, H), jnp.float32),
                   jax.ShapeDtypeStruct((n, H), jnp.float32)],
    )


@functools.cache
def _head_call(n, bm=1000):
    grid = (n // bm,)

    def body(slo_ref, shi_ref, cnt_ref, w_ref, b_ref, o_ref):
        seg = jnp.concatenate([slo_ref[...], shi_ref[...]], axis=1)
        mean = seg / jnp.maximum(cnt_ref[...][:, 0:1], 1.0)
        raw = jnp.dot(mean, w_ref[...],
                      preferred_element_type=jnp.float32) + b_ref[...]
        lane = lax.broadcasted_iota(jnp.int32, (bm, 128), 1)
        o_ref[...] = jnp.where(lane == 1, jax.nn.sigmoid(raw), raw)

    return pl.pallas_call(
        body,
        grid=grid,
        in_specs=[
            pl.BlockSpec((bm, H), lambda i: (i, 0)),
            pl.BlockSpec((bm, H), lambda i: (i, 0)),
            pl.BlockSpec((bm, 16), lambda i: (i, 0)),
            pl.BlockSpec((D, 128), lambda i: (0, 0)),
            pl.BlockSpec((1, 128), lambda i: (0, 0)),
        ],
        out_specs=pl.BlockSpec((bm, 128), lambda i: (i, 0)),
        out_shape=jax.ShapeDtypeStruct((n, 128), jnp.float32),
    )


def kernel(x_tad, x_atac_region, x_gene, x_protein, edge_index_tad__overlaps__atac_region, edge_index_atac_region__rev_overlaps__tad, edge_index_tad__overlaps__gene, edge_index_gene__rev_overlaps__tad, edge_index_atac_region__overlaps__gene, edge_index_gene__rev_overlaps__atac_region, edge_index_protein__coexpressed__protein, edge_index_protein__tf_interacts__gene, edge_index_gene__rev_tf_interacts__protein, edge_index_protein__rev_associated__gene, edge_index_gene__associated__protein, edge_index_protein__is_named__gene_name, params):
    kw = dict(locals())
    xs = {nt: kw["x_" + nt] for nt in NODE_TYPES}
    eis = {_rk(r): kw["edge_index_" + _rk(r)] for r in RELS + [NAME_REL]}

    zerosH = jnp.zeros((64, H), jnp.float32)
    onesH = jnp.ones((CHUNK, H), jnp.float32)
    ones16 = jnp.ones((CHUNK, 16), jnp.float32)
    zeros16 = jnp.zeros((64, 16), jnp.float32)

    bins, cnts = {}, {}
    for r in RELS + [NAME_REL]:
        k = _rk(r)
        ei = eis[k]
        srcs_b, ldst_b, nch16, basech16, NW = _bin_edges(
            ei[0], ei[1], NODE_N[r[2]], NODE_N[r[0]])
        bins[k] = (srcs_b, ldst_b, nch16, basech16, NW)
        cnts[k] = _counts_call(srcs_b.shape[0], NW)(
            ldst_b, nch16, basech16, ones16, zeros16)

    x = {t: (xs[t][:, 0:H], xs[t][:, H:D]) for t in NODE_TYPES}
    for l in range(N_LAYERS):
        lp = params["conv%d" % l]
        segs = {}
        for r in RELS:
            k = _rk(r)
            srcs_b, ldst_b, nch16, basech16, NW = bins[k]
            call = _segsum_call(NODE_N[r[0]], srcs_b.shape[0], NW)
            segs[k] = tuple(
                call(x[r[0]][h], srcs_b, ldst_b, nch16, basech16, zerosH)
                for h in range(2))
        new = {}
        for dst_t in NODE_TYPES:
            rels_t = [r for r in RELS if r[2] == dst_t]
            ks = [_rk(r) for r in rels_t]
            wr_sum = sum(lp[k]["W_r"] for k in ks)
            wcat = jnp.concatenate([wr_sum] + [lp[k]["W_l"] for k in ks], axis=0)
            bias = sum(lp[k]["b_l"] for k in ks).reshape(1, D)
            n = NODE_N[dst_t]
            args = [x[dst_t][0], x[dst_t][1]]
            for k in ks:
                args.append(segs[k][0])
                args.append(segs[k][1])
                args.append(cnts[k])
            new[dst_t] = tuple(_conv_call(n, len(ks))(*args, wcat, bias))
        x = new

    kn = _rk(NAME_REL)
    srcs_b, ldst_b, nch16, basech16, NW = bins[kn]
    call = _segsum_call(NODE_N["protein"], srcs_b.shape[0], NW)
    seg_n = tuple(
        call(x["protein"][h], srcs_b, ldst_b, nch16, basech16, zerosH)
        for h in range(2))
    p1, p2 = params["name_conv"], params["zero_conv"]
    w2 = jnp.zeros((D, 128), jnp.float32)
    w2 = w2.at[:, 0].set(p1["W_l"][:, 0]).at[:, 1].set(p2["W_l"][:, 0])
    b2 = jnp.zeros((1, 128), jnp.float32)
    b2 = b2.at[0, 0].set(p1["b_l"][0] - p1["W_r"][0, 0])
    b2 = b2.at[0, 1].set(p2["b_l"][0] - p2["W_r"][0, 0])
    outh = _head_call(NODE_N["gene_name"])(seg_n[0], seg_n[1], cnts[kn], w2, b2)
    return outh[:, :2]
